# baseline (speedup 1.0000x reference)
import functools

import jax
import jax.numpy as jnp
from jax import lax
from jax.experimental import pallas as pl
from jax.experimental.pallas import tpu as pltpu
from jax.experimental.pallas import tpu_sc as plsc


def _gather_call(B, V, D, b_per_w, NC):
    mesh = plsc.VectorSubcoreMesh(core_axis_name="c", subcore_axis_name="s")
    lanes = 16
    half = 8
    n_batches = b_per_w // half
    tail_start = (V // 128) * 128
    tail_len = V - tail_start
    max_full_base = tail_start - 128

    @functools.partial(
        pl.kernel,
        mesh=mesh,
        out_type=jax.ShapeDtypeStruct((D, B), jnp.float32),
        scratch_types=[
            pltpu.VMEM((b_per_w,), jnp.int32),
            pltpu.VMEM((3 * half, D, 128), jnp.float32),
            pltpu.VMEM((tail_len, D), jnp.float32),
            pltpu.VMEM((D, b_per_w), jnp.float32),
            pltpu.SemaphoreType.DMA,
            pltpu.SemaphoreType.DMA,
            pltpu.SemaphoreType.DMA,
        ],
        compiler_params=pltpu.CompilerParams(needs_layout_passes=False),
    )
    def k(labels_hbm, table_hbm, tail_hbm, out_hbm, lab_v, cols_v, tail_v,
          out_v, sem0, sem1, sem2):
        sems = (sem0, sem1, sem2)
        wid = lax.axis_index("s") * NC + lax.axis_index("c")
        base = wid * b_per_w
        pltpu.sync_copy(labels_hbm.at[pl.ds(base, b_per_w)], lab_v)
        pltpu.sync_copy(tail_hbm, tail_v)

        lane_ids = lax.iota(jnp.int32, lanes)

        def fire(b, ring, sem_ix):
            labs16 = lab_v[pl.ds((b // 2) * lanes, lanes)]
            h = (b % 2) * half
            for i in range(half):
                lab = jnp.max(jnp.where(lane_ids == h + i, labs16, 0))
                colb = pl.multiple_of(
                    jnp.minimum(lab & (-128), max_full_base), 128
                )
                pltpu.async_copy(
                    table_hbm.at[:, pl.ds(colb, 128)],
                    cols_v.at[ring * half + i],
                    sems[sem_ix],
                )

        def drain(ring, sem_ix):
            for i in range(half):
                pltpu.make_async_copy(
                    table_hbm.at[:, pl.ds(0, 128)],
                    cols_v.at[ring * half + i],
                    sems[sem_ix],
                ).wait()

        def extract(pair, s0, s1):
            labs = lab_v[pl.ds(pair * lanes, lanes)]
            lane_vec = labs & 127
            in_tail = labs >= tail_start
            tail_idx = jnp.clip(labs - tail_start, 0, tail_len - 1)
            slot_vec = jnp.where(
                lane_ids < half,
                s0 * half + lane_ids,
                s1 * half + (lane_ids - half),
            )
            for j in range(D):
                j_vec = jnp.full((lanes,), j, dtype=jnp.int32)
                vals = plsc.load_gather(cols_v, [slot_vec, j_vec, lane_vec])
                tvals = plsc.load_gather(tail_v, [tail_idx, j_vec])
                out_v[j, pl.ds(pair * lanes, lanes)] = jnp.where(
                    in_tail, tvals, vals
                )

        fire(0, 0, 0)

        def body(b):
            for s in range(3):
                @pl.when(b % 3 == s)
                def _():
                    fire(b, s, s)
                @pl.when((b - 1) % 3 == s)
                def _():
                    drain(s, s)

            @pl.when(b % 2 == 0)
            def _():
                extract((b - 2) // 2, (b - 2) % 3, (b - 1) % 3)

        pl.loop(1, n_batches)(body)
        drain((n_batches - 1) % 3, (n_batches - 1) % 3)
        extract(n_batches // 2 - 1, (n_batches - 2) % 3, (n_batches - 1) % 3)

        pltpu.sync_copy(out_v, out_hbm.at[:, pl.ds(base, b_per_w)])

    return k


def kernel(labels, embedding):
    (B,) = labels.shape
    V, D = embedding.shape
    info = plsc.get_sparse_core_info()
    NC, NS = info.num_cores, info.num_subcores
    NW = NC * NS
    b_per_w = B // NW
    tail_start = (V // 128) * 128
    call = _gather_call(B, V, D, b_per_w, NC)
    out_t = call(
        labels.astype(jnp.int32), embedding.T, embedding[tail_start:, :]
    )
    return out_t.T


# speedup vs baseline: 1.1647x; 1.1647x over previous
import functools

import jax
import jax.numpy as jnp
from jax import lax
from jax.experimental import pallas as pl
from jax.experimental.pallas import tpu as pltpu
from jax.experimental.pallas import tpu_sc as plsc


def _gather_call(B, V, D, b_per_w, NC):
    mesh = plsc.VectorSubcoreMesh(core_axis_name="c", subcore_axis_name="s")
    lanes = 16
    n_groups = b_per_w // lanes
    tail_start = (V // 128) * 128
    tail_len = V - tail_start
    max_full_base = tail_start - 128

    @functools.partial(
        pl.kernel,
        mesh=mesh,
        out_type=jax.ShapeDtypeStruct((D, B), jnp.float32),
        scratch_types=[
            pltpu.VMEM((b_per_w,), jnp.int32),
            pltpu.VMEM((lanes, D, 128), jnp.float32),
            pltpu.VMEM((tail_len, D), jnp.float32),
            pltpu.VMEM((D, b_per_w), jnp.float32),
            pltpu.SemaphoreType.DMA,
        ],
        compiler_params=pltpu.CompilerParams(needs_layout_passes=False),
    )
    def k(labels_hbm, table_hbm, tail_hbm, out_hbm, lab_v, cols_v, tail_v, out_v, sem):
        wid = lax.axis_index("s") * NC + lax.axis_index("c")
        base = wid * b_per_w
        pltpu.sync_copy(labels_hbm.at[pl.ds(base, b_per_w)], lab_v)
        pltpu.sync_copy(tail_hbm, tail_v)

        lane_ids = lax.iota(jnp.int32, lanes)

        def group(g):
            labs = lab_v[pl.ds(g * lanes, lanes)]
            lane_vec = labs & 127
            in_tail = labs >= tail_start
            tail_idx = jnp.clip(labs - tail_start, 0, tail_len - 1)
            for i in range(lanes):
                lab = jnp.max(jnp.where(lane_ids == i, labs, 0))
                colb = pl.multiple_of(
                    jnp.minimum(lab & (-128), max_full_base), 128
                )
                pltpu.async_copy(
                    table_hbm.at[:, pl.ds(colb, 128)], cols_v.at[i], sem
                )
            for i in range(lanes):
                pltpu.make_async_copy(
                    table_hbm.at[:, pl.ds(0, 128)], cols_v.at[i], sem
                ).wait()
            for j in range(D):
                j_vec = jnp.full((lanes,), j, dtype=jnp.int32)
                vals = plsc.load_gather(cols_v, [lane_ids, j_vec, lane_vec])
                tvals = plsc.load_gather(tail_v, [tail_idx, j_vec])
                out_v[j, pl.ds(g * lanes, lanes)] = jnp.where(in_tail, tvals, vals)

        pl.loop(0, n_groups)(group)
        pltpu.sync_copy(out_v, out_hbm.at[:, pl.ds(base, b_per_w)])

    return k


def kernel(labels, embedding):
    (B,) = labels.shape
    V, D = embedding.shape
    info = plsc.get_sparse_core_info()
    NC, NS = info.num_cores, info.num_subcores
    NW = NC * NS
    b_per_w = B // NW
    tail_start = (V // 128) * 128
    call = _gather_call(B, V, D, b_per_w, NC)
    out_t = call(
        labels.astype(jnp.int32), embedding.T, embedding[tail_start:, :]
    )
    return out_t.T
